# speedup vs baseline: 3.3633x; 3.3633x over previous
import jax
import jax.numpy as jnp
from jax.experimental import pallas as pl
from jax.experimental.pallas import tpu as pltpu

K = 5
W = 0.2
EPS = 1e-8

N = 100000
D = 128
B = 4096
C = 1000

ROWS_PER_STEP = 800
N_STEPS = N // ROWS_PER_STEP
CE_ROWS = 512
NEG_INF = float("-inf")


def _ce_kernel(p_ref, lab_ref, out_ref):
    i = pl.program_id(0)
    p = p_ref[...]
    lab = lab_ref[...]
    rowmax = jnp.max(p, axis=1, keepdims=True)
    lse = jnp.log(jnp.sum(jnp.exp(p - rowmax), axis=1, keepdims=True)) + rowmax
    cols = jax.lax.broadcasted_iota(jnp.int32, p.shape, 1)
    sel = jnp.sum(jnp.where(cols == lab, p, 0.0), axis=1, keepdims=True)
    nll_sum = jnp.sum(lse - sel)

    @pl.when(i == 0)
    def _init():
        out_ref[0, 0] = 0.0

    out_ref[0, 0] += nll_sum


def _cos_kernel(x_ref, tf_ref, cos_ref):
    x = x_ref[...]
    xn = x * jax.lax.rsqrt(jnp.maximum(jnp.sum(x * x), EPS * EPS))
    q = tf_ref[...]
    dims = (((1,), (1,)), ((), ()))
    raw = jax.lax.dot_general(xn, q, dims,
                              preferred_element_type=jnp.float32)
    ones = jnp.ones((1, D), dtype=jnp.float32)
    nrm2 = jax.lax.dot_general(ones, q * q, dims,
                               preferred_element_type=jnp.float32)
    inv = 1.0 / jnp.maximum(jnp.sqrt(nrm2), EPS)
    cos_ref[...] = (raw * inv)[:, None, :]


def _select_kernel(cos_ref, lab_ref, xl_ref, ce_ref, out_ref):
    cos = cos_ref[...]
    lab = lab_ref[...]
    xl = xl_ref[0]
    pos = lab == xl

    ps = jnp.where(pos, cos, NEG_INF)
    pos_sum = jnp.float32(0.0)
    for _ in range(K):
        m = jnp.max(ps)
        pos_sum += m
        ps = jnp.where(ps >= m, NEG_INF, ps)

    ns = jnp.where(pos, NEG_INF, 1.0 / cos)
    den = jnp.float32(0.0)
    for _ in range(K):
        v = jnp.max(ns)
        den += jnp.exp(1.0 / v)
        ns = jnp.where(ns >= v, NEG_INF, ns)

    contrastive = (-1.0 / (2.0 * K)) * (pos_sum - K * jnp.log(den))
    ce = ce_ref[0] / jnp.float32(B)
    out_ref[0] = ce * (1.0 - W) + contrastive * W


def kernel(label, predict, x_feature, x_label, train_features, train_labels):
    ce_sum = pl.pallas_call(
        _ce_kernel,
        grid=(B // CE_ROWS,),
        in_specs=[
            pl.BlockSpec((CE_ROWS, C), lambda i: (i, 0)),
            pl.BlockSpec((CE_ROWS, 1), lambda i: (i, 0)),
        ],
        out_specs=pl.BlockSpec(memory_space=pltpu.SMEM),
        out_shape=jax.ShapeDtypeStruct((1, 1), jnp.float32),
    )(predict, label.astype(jnp.int32).reshape(B, 1))

    cos = pl.pallas_call(
        _cos_kernel,
        grid=(N_STEPS,),
        in_specs=[
            pl.BlockSpec((1, D), lambda i: (0, 0)),
            pl.BlockSpec((ROWS_PER_STEP, D), lambda i: (i, 0)),
        ],
        out_specs=pl.BlockSpec((1, 1, ROWS_PER_STEP), lambda i: (i, 0, 0)),
        out_shape=jax.ShapeDtypeStruct((N_STEPS, 1, ROWS_PER_STEP), jnp.float32),
    )(x_feature.reshape(1, D), train_features)

    loss = pl.pallas_call(
        _select_kernel,
        in_specs=[
            pl.BlockSpec((N_STEPS, ROWS_PER_STEP), lambda: (0, 0)),
            pl.BlockSpec((N_STEPS, ROWS_PER_STEP), lambda: (0, 0)),
            pl.BlockSpec(memory_space=pltpu.SMEM),
            pl.BlockSpec(memory_space=pltpu.SMEM),
        ],
        out_specs=pl.BlockSpec(memory_space=pltpu.SMEM),
        out_shape=jax.ShapeDtypeStruct((1,), jnp.float32),
    )(
        cos.reshape(N_STEPS, ROWS_PER_STEP),
        train_labels.astype(jnp.int32).reshape(N_STEPS, ROWS_PER_STEP),
        jnp.asarray(x_label, jnp.int32).reshape(1),
        ce_sum.reshape(1),
    )
    return loss[0]


# speedup vs baseline: 6.3164x; 1.8781x over previous
import jax
import jax.numpy as jnp
from jax.experimental import pallas as pl
from jax.experimental.pallas import tpu as pltpu

K = 5
W = 0.2
EPS = 1e-8

N = 100000
D = 128
B = 4096
C = 1000

ROWS_PER_STEP = 4000
N_STEPS = N // ROWS_PER_STEP
CE_ROWS = 1024
NEG_INF = float("-inf")


def _ce_kernel(p_ref, lab_ref, out_ref):
    i = pl.program_id(0)
    p = p_ref[...]
    lab = lab_ref[...]
    rowmax = jnp.max(p, axis=1, keepdims=True)
    lse = jnp.log(jnp.sum(jnp.exp(p - rowmax), axis=1, keepdims=True)) + rowmax
    cols = jax.lax.broadcasted_iota(jnp.int32, p.shape, 1)
    sel = jnp.sum(jnp.where(cols == lab, p, 0.0), axis=1, keepdims=True)
    nll_sum = jnp.sum(lse - sel)

    @pl.when(i == 0)
    def _init():
        out_ref[0, 0] = 0.0

    out_ref[0, 0] += nll_sum


def _cos_kernel(x_ref, tf_ref, cos_ref):
    x = x_ref[...]
    xn = x * jax.lax.rsqrt(jnp.maximum(jnp.sum(x * x), EPS * EPS))
    q = tf_ref[...]
    dims = (((1,), (1,)), ((), ()))
    raw = jax.lax.dot_general(xn, q, dims,
                              preferred_element_type=jnp.float32)
    ones = jnp.ones((1, D), dtype=jnp.float32)
    nrm2 = jax.lax.dot_general(ones, q * q, dims,
                               preferred_element_type=jnp.float32)
    inv = 1.0 / jnp.maximum(jnp.sqrt(nrm2), EPS)
    cos_ref[...] = (raw * inv)[:, None, :]


def _select_kernel(cos_ref, lab_ref, xl_ref, ce_ref, out_ref):
    cos = cos_ref[...]
    lab = lab_ref[...]
    xl = xl_ref[0]
    pos = lab == xl

    ps = jnp.where(pos, cos, NEG_INF)
    pos_sum = jnp.float32(0.0)
    for _ in range(K):
        m = jnp.max(ps)
        pos_sum += m
        ps = jnp.where(ps >= m, NEG_INF, ps)

    ns = jnp.where(pos, NEG_INF, 1.0 / cos)
    den = jnp.float32(0.0)
    for _ in range(K):
        v = jnp.max(ns)
        den += jnp.exp(1.0 / v)
        ns = jnp.where(ns >= v, NEG_INF, ns)

    contrastive = (-1.0 / (2.0 * K)) * (pos_sum - K * jnp.log(den))
    ce = ce_ref[0] / jnp.float32(B)
    out_ref[0] = ce * (1.0 - W) + contrastive * W


def kernel(label, predict, x_feature, x_label, train_features, train_labels):
    ce_sum = pl.pallas_call(
        _ce_kernel,
        grid=(B // CE_ROWS,),
        in_specs=[
            pl.BlockSpec((CE_ROWS, C), lambda i: (i, 0)),
            pl.BlockSpec((CE_ROWS, 1), lambda i: (i, 0)),
        ],
        out_specs=pl.BlockSpec(memory_space=pltpu.SMEM),
        out_shape=jax.ShapeDtypeStruct((1, 1), jnp.float32),
    )(predict, label.astype(jnp.int32).reshape(B, 1))

    cos = pl.pallas_call(
        _cos_kernel,
        grid=(N_STEPS,),
        in_specs=[
            pl.BlockSpec((1, D), lambda i: (0, 0)),
            pl.BlockSpec((ROWS_PER_STEP, D), lambda i: (i, 0)),
        ],
        out_specs=pl.BlockSpec((1, 1, ROWS_PER_STEP), lambda i: (i, 0, 0)),
        out_shape=jax.ShapeDtypeStruct((N_STEPS, 1, ROWS_PER_STEP), jnp.float32),
    )(x_feature.reshape(1, D), train_features)

    loss = pl.pallas_call(
        _select_kernel,
        in_specs=[
            pl.BlockSpec((N_STEPS, ROWS_PER_STEP), lambda: (0, 0)),
            pl.BlockSpec((N_STEPS, ROWS_PER_STEP), lambda: (0, 0)),
            pl.BlockSpec(memory_space=pltpu.SMEM),
            pl.BlockSpec(memory_space=pltpu.SMEM),
        ],
        out_specs=pl.BlockSpec(memory_space=pltpu.SMEM),
        out_shape=jax.ShapeDtypeStruct((1,), jnp.float32),
    )(
        cos.reshape(N_STEPS, ROWS_PER_STEP),
        train_labels.astype(jnp.int32).reshape(N_STEPS, ROWS_PER_STEP),
        jnp.asarray(x_label, jnp.int32).reshape(1),
        ce_sum.reshape(1),
    )
    return loss[0]


# speedup vs baseline: 7.2453x; 1.1471x over previous
import jax
import jax.numpy as jnp
from jax.experimental import pallas as pl
from jax.experimental.pallas import tpu as pltpu

K = 5
W = 0.2
EPS = 1e-8

N = 100000
D = 128
B = 4096
C = 1000

ROWS_PER_STEP = 10000
N_STEPS = N // ROWS_PER_STEP
CE_ROWS = 2048
NEG_INF = float("-inf")


def _ce_kernel(p_ref, lab_ref, out_ref):
    i = pl.program_id(0)
    p = p_ref[...]
    lab = lab_ref[...]
    rowmax = jnp.max(p, axis=1, keepdims=True)
    lse = jnp.log(jnp.sum(jnp.exp(p - rowmax), axis=1, keepdims=True)) + rowmax
    cols = jax.lax.broadcasted_iota(jnp.int32, p.shape, 1)
    sel = jnp.sum(jnp.where(cols == lab, p, 0.0), axis=1, keepdims=True)
    nll_sum = jnp.sum(lse - sel)

    @pl.when(i == 0)
    def _init():
        out_ref[0, 0] = 0.0

    out_ref[0, 0] += nll_sum


def _cos_kernel(x_ref, tf_ref, cos_ref):
    x = x_ref[...]
    xn = x * jax.lax.rsqrt(jnp.maximum(jnp.sum(x * x), EPS * EPS))
    q = tf_ref[...]
    dims = (((1,), (1,)), ((), ()))
    raw = jax.lax.dot_general(xn, q, dims,
                              preferred_element_type=jnp.float32)
    ones = jnp.ones((1, D), dtype=jnp.float32)
    nrm2 = jax.lax.dot_general(ones, q * q, dims,
                               preferred_element_type=jnp.float32)
    inv = 1.0 / jnp.maximum(jnp.sqrt(nrm2), EPS)
    cos_ref[...] = (raw * inv)[:, None, :]


def _select_kernel(cos_ref, lab_ref, xl_ref, ce_ref, out_ref):
    cos = cos_ref[...]
    lab = lab_ref[...]
    xl = xl_ref[0]
    pos = lab == xl

    ps = jnp.where(pos, cos, NEG_INF)
    pos_sum = jnp.float32(0.0)
    for _ in range(K):
        m = jnp.max(ps)
        pos_sum += m
        ps = jnp.where(ps >= m, NEG_INF, ps)

    ns = jnp.where(pos, NEG_INF, 1.0 / cos)
    den = jnp.float32(0.0)
    for _ in range(K):
        v = jnp.max(ns)
        den += jnp.exp(1.0 / v)
        ns = jnp.where(ns >= v, NEG_INF, ns)

    contrastive = (-1.0 / (2.0 * K)) * (pos_sum - K * jnp.log(den))
    ce = ce_ref[0] / jnp.float32(B)
    out_ref[0] = ce * (1.0 - W) + contrastive * W


def kernel(label, predict, x_feature, x_label, train_features, train_labels):
    ce_sum = pl.pallas_call(
        _ce_kernel,
        grid=(B // CE_ROWS,),
        in_specs=[
            pl.BlockSpec((CE_ROWS, C), lambda i: (i, 0)),
            pl.BlockSpec((CE_ROWS, 1), lambda i: (i, 0)),
        ],
        out_specs=pl.BlockSpec(memory_space=pltpu.SMEM),
        out_shape=jax.ShapeDtypeStruct((1, 1), jnp.float32),
    )(predict, label.astype(jnp.int32).reshape(B, 1))

    cos = pl.pallas_call(
        _cos_kernel,
        grid=(N_STEPS,),
        in_specs=[
            pl.BlockSpec((1, D), lambda i: (0, 0)),
            pl.BlockSpec((ROWS_PER_STEP, D), lambda i: (i, 0)),
        ],
        out_specs=pl.BlockSpec((1, 1, ROWS_PER_STEP), lambda i: (i, 0, 0)),
        out_shape=jax.ShapeDtypeStruct((N_STEPS, 1, ROWS_PER_STEP), jnp.float32),
    )(x_feature.reshape(1, D), train_features)

    loss = pl.pallas_call(
        _select_kernel,
        in_specs=[
            pl.BlockSpec((N_STEPS, ROWS_PER_STEP), lambda: (0, 0)),
            pl.BlockSpec((N_STEPS, ROWS_PER_STEP), lambda: (0, 0)),
            pl.BlockSpec(memory_space=pltpu.SMEM),
            pl.BlockSpec(memory_space=pltpu.SMEM),
        ],
        out_specs=pl.BlockSpec(memory_space=pltpu.SMEM),
        out_shape=jax.ShapeDtypeStruct((1,), jnp.float32),
    )(
        cos.reshape(N_STEPS, ROWS_PER_STEP),
        train_labels.astype(jnp.int32).reshape(N_STEPS, ROWS_PER_STEP),
        jnp.asarray(x_label, jnp.int32).reshape(1),
        ce_sum.reshape(1),
    )
    return loss[0]


# speedup vs baseline: 7.4781x; 1.0321x over previous
import jax
import jax.numpy as jnp
from jax.experimental import pallas as pl
from jax.experimental.pallas import tpu as pltpu

K = 5
W = 0.2
EPS = 1e-8

N = 100000
D = 128
B = 4096
C = 1000

ROWS_PER_STEP = 20000
N_STEPS = N // ROWS_PER_STEP
CE_ROWS = 2048
NEG_INF = float("-inf")


def _ce_kernel(p_ref, lab_ref, out_ref):
    i = pl.program_id(0)
    p = p_ref[...]
    lab = lab_ref[...]
    rowmax = jnp.max(p, axis=1, keepdims=True)
    lse = jnp.log(jnp.sum(jnp.exp(p - rowmax), axis=1, keepdims=True)) + rowmax
    cols = jax.lax.broadcasted_iota(jnp.int32, p.shape, 1)
    sel = jnp.sum(jnp.where(cols == lab, p, 0.0), axis=1, keepdims=True)
    nll_sum = jnp.sum(lse - sel)

    @pl.when(i == 0)
    def _init():
        out_ref[0, 0] = 0.0

    out_ref[0, 0] += nll_sum


def _cos_kernel(x_ref, tf_ref, cos_ref):
    x = x_ref[...]
    xn = x * jax.lax.rsqrt(jnp.maximum(jnp.sum(x * x), EPS * EPS))
    q = tf_ref[...]
    dims = (((1,), (1,)), ((), ()))
    raw = jax.lax.dot_general(xn, q, dims,
                              preferred_element_type=jnp.float32)
    ones = jnp.ones((1, D), dtype=jnp.float32)
    nrm2 = jax.lax.dot_general(ones, q * q, dims,
                               preferred_element_type=jnp.float32)
    inv = 1.0 / jnp.maximum(jnp.sqrt(nrm2), EPS)
    cos_ref[...] = (raw * inv)[:, None, :]


def _select_kernel(cos_ref, lab_ref, xl_ref, ce_ref, out_ref):
    cos = cos_ref[...]
    lab = lab_ref[...]
    xl = xl_ref[0]
    pos = lab == xl

    ps = jnp.where(pos, cos, NEG_INF)
    pos_sum = jnp.float32(0.0)
    for _ in range(K):
        m = jnp.max(ps)
        pos_sum += m
        ps = jnp.where(ps >= m, NEG_INF, ps)

    ns = jnp.where(pos, NEG_INF, 1.0 / cos)
    den = jnp.float32(0.0)
    for _ in range(K):
        v = jnp.max(ns)
        den += jnp.exp(1.0 / v)
        ns = jnp.where(ns >= v, NEG_INF, ns)

    contrastive = (-1.0 / (2.0 * K)) * (pos_sum - K * jnp.log(den))
    ce = ce_ref[0] / jnp.float32(B)
    out_ref[0] = ce * (1.0 - W) + contrastive * W


def kernel(label, predict, x_feature, x_label, train_features, train_labels):
    ce_sum = pl.pallas_call(
        _ce_kernel,
        grid=(B // CE_ROWS,),
        in_specs=[
            pl.BlockSpec((CE_ROWS, C), lambda i: (i, 0)),
            pl.BlockSpec((CE_ROWS, 1), lambda i: (i, 0)),
        ],
        out_specs=pl.BlockSpec(memory_space=pltpu.SMEM),
        out_shape=jax.ShapeDtypeStruct((1, 1), jnp.float32),
    )(predict, label.astype(jnp.int32).reshape(B, 1))

    cos = pl.pallas_call(
        _cos_kernel,
        grid=(N_STEPS,),
        in_specs=[
            pl.BlockSpec((1, D), lambda i: (0, 0)),
            pl.BlockSpec((ROWS_PER_STEP, D), lambda i: (i, 0)),
        ],
        out_specs=pl.BlockSpec((1, 1, ROWS_PER_STEP), lambda i: (i, 0, 0)),
        out_shape=jax.ShapeDtypeStruct((N_STEPS, 1, ROWS_PER_STEP), jnp.float32),
    )(x_feature.reshape(1, D), train_features)

    loss = pl.pallas_call(
        _select_kernel,
        in_specs=[
            pl.BlockSpec((N_STEPS, ROWS_PER_STEP), lambda: (0, 0)),
            pl.BlockSpec((N_STEPS, ROWS_PER_STEP), lambda: (0, 0)),
            pl.BlockSpec(memory_space=pltpu.SMEM),
            pl.BlockSpec(memory_space=pltpu.SMEM),
        ],
        out_specs=pl.BlockSpec(memory_space=pltpu.SMEM),
        out_shape=jax.ShapeDtypeStruct((1,), jnp.float32),
    )(
        cos.reshape(N_STEPS, ROWS_PER_STEP),
        train_labels.astype(jnp.int32).reshape(N_STEPS, ROWS_PER_STEP),
        jnp.asarray(x_label, jnp.int32).reshape(1),
        ce_sum.reshape(1),
    )
    return loss[0]
